# Initial kernel scaffold; baseline (speedup 1.0000x reference)
import jax
import jax.numpy as jnp
from jax.experimental import pallas as pl


def kernel(x, gate_W, up_W, down_W, router_W, router_b, gate_A, gate_B, up_A, up_B, down_A, down_B):
    raise NotImplementedError("write your pallas kernel here")


# speedup vs baseline: 1.1020x; 1.1020x over previous
import functools

import jax
import jax.numpy as jnp
from jax import lax
from jax.experimental import pallas as pl
from jax.experimental.pallas import tpu as pltpu

_ALPHA = 32
_RANK = 16


def _nt_dot(a, b):
    return lax.dot_general(a, b, (((1,), (1,)), ((), ())),
                           preferred_element_type=jnp.float32)


def _nn_dot(a, b):
    return lax.dot_general(a, b, (((1,), (0,)), ((), ())),
                           preferred_element_type=jnp.float32)


def _router_body(x_ref, w_ref, b_ref, rout_ref, ec_ref):
    x = x_ref[...]
    w = w_ref[...]
    logits = _nt_dot(x, w) + b_ref[0:1, :]
    mx = jnp.max(logits, axis=-1, keepdims=True)
    e = jnp.exp(logits - mx)
    routing = e / jnp.sum(e, axis=-1, keepdims=True)
    rmax = jnp.max(routing, axis=-1, keepdims=True)
    lane = lax.broadcasted_iota(jnp.int32, routing.shape, 1)
    first = jnp.min(jnp.where(routing == rmax, lane, routing.shape[-1]),
                    axis=-1, keepdims=True)
    y_hard = (lane == first).astype(jnp.float32)
    rout_ref[...] = routing
    ec_ref[...] = (y_hard - routing) + routing


def _moe_body(nm, scaling,
              xbf_ref, rout_ref, gw_ref, uw_ref, dw_ref,
              bg_ref, bu_ref, ad_ref, ag_ref, au_ref, bd_ref,
              out_ref,
              hwg_ref, hwu_ref, had_ref, rrep_ref):
    m = pl.program_id(1)
    er = ag_ref.shape[0]

    @pl.when(m == 0)
    def _init():
        routing = rout_ref[...]
        n_e = routing.shape[1]
        r0 = lax.broadcasted_iota(jnp.int32, (n_e, er), 0)
        r1 = lax.broadcasted_iota(jnp.int32, (n_e, er), 1)
        expand = (r1 // _RANK == r0).astype(jnp.float32)
        rrep = _nn_dot(routing, expand)
        rrep_ref[...] = rrep
        xbf = xbf_ref[...]
        xag = _nt_dot(xbf, ag_ref[...].astype(jnp.bfloat16))
        xau = _nt_dot(xbf, au_ref[...].astype(jnp.bfloat16))
        hwg_ref[...] = (xag * rrep * scaling).astype(jnp.bfloat16)
        hwu_ref[...] = (xau * rrep * scaling).astype(jnp.bfloat16)

    xbf = xbf_ref[...]
    g = _nt_dot(xbf, gw_ref[...].astype(jnp.bfloat16))
    u = _nt_dot(xbf, uw_ref[...].astype(jnp.bfloat16))
    g = g + _nn_dot(hwg_ref[...], bg_ref[...].astype(jnp.bfloat16))
    u = u + _nn_dot(hwu_ref[...], bu_ref[...].astype(jnp.bfloat16))
    h = (g * jax.nn.sigmoid(g)) * u
    hbf = h.astype(jnp.bfloat16)
    dcontrib = _nt_dot(hbf, dw_ref[...].astype(jnp.bfloat16))
    acontrib = _nt_dot(hbf, ad_ref[...].astype(jnp.bfloat16))

    @pl.when(m == 0)
    def _first():
        out_ref[...] = dcontrib
        had_ref[...] = acontrib

    @pl.when(m > 0)
    def _acc():
        out_ref[...] += dcontrib
        had_ref[...] += acontrib

    @pl.when(m == nm - 1)
    def _fin():
        hwd = (had_ref[...] * rrep_ref[...] * scaling).astype(jnp.bfloat16)
        out_ref[...] += _nn_dot(hwd, bd_ref[...].astype(jnp.bfloat16))


def kernel(x, gate_W, up_W, down_W, router_W, router_b,
           gate_A, gate_B, up_A, up_B, down_A, down_B):
    b, s, d = x.shape
    m_dim = gate_W.shape[0]
    n_e, rank, _ = gate_A.shape
    er = n_e * rank
    n = b * s
    scaling = _ALPHA / _RANK

    x2 = x.reshape(n, d)
    t_r = min(2048, n)
    nt_r = n // t_r
    routing, ec = pl.pallas_call(
        _router_body,
        grid=(nt_r,),
        in_specs=[
            pl.BlockSpec((t_r, d), lambda t: (t, 0)),
            pl.BlockSpec((n_e, d), lambda t: (0, 0)),
            pl.BlockSpec((8, n_e), lambda t: (0, 0)),
        ],
        out_specs=[
            pl.BlockSpec((t_r, n_e), lambda t: (t, 0)),
            pl.BlockSpec((t_r, n_e), lambda t: (t, 0)),
        ],
        out_shape=[jax.ShapeDtypeStruct((n, n_e), jnp.float32)] * 2,
    )(x2, router_W, jnp.broadcast_to(router_b.reshape(1, n_e), (8, n_e)))

    xbf = x2.astype(jnp.bfloat16)
    bg = gate_B.transpose(0, 2, 1).reshape(er, m_dim)
    bu = up_B.transpose(0, 2, 1).reshape(er, m_dim)
    ad = down_A.reshape(er, m_dim)
    ag = gate_A.reshape(er, d)
    au = up_A.reshape(er, d)
    bd = down_B.transpose(0, 2, 1).reshape(er, d)

    t = min(1024, n)
    mt = min(256, m_dim)
    nt = n // t
    nm = m_dim // mt

    out = pl.pallas_call(
        functools.partial(_moe_body, nm, scaling),
        grid=(nt, nm),
        in_specs=[
            pl.BlockSpec((t, d), lambda i, j: (i, 0)),
            pl.BlockSpec((t, n_e), lambda i, j: (i, 0)),
            pl.BlockSpec((mt, d), lambda i, j: (j, 0)),
            pl.BlockSpec((mt, d), lambda i, j: (j, 0)),
            pl.BlockSpec((d, mt), lambda i, j: (0, j)),
            pl.BlockSpec((er, mt), lambda i, j: (0, j)),
            pl.BlockSpec((er, mt), lambda i, j: (0, j)),
            pl.BlockSpec((er, mt), lambda i, j: (0, j)),
            pl.BlockSpec((er, d), lambda i, j: (0, 0)),
            pl.BlockSpec((er, d), lambda i, j: (0, 0)),
            pl.BlockSpec((er, d), lambda i, j: (0, 0)),
        ],
        out_specs=pl.BlockSpec((t, d), lambda i, j: (i, 0)),
        out_shape=jax.ShapeDtypeStruct((n, d), jnp.float32),
        scratch_shapes=[
            pltpu.VMEM((t, er), jnp.bfloat16),
            pltpu.VMEM((t, er), jnp.bfloat16),
            pltpu.VMEM((t, er), jnp.float32),
            pltpu.VMEM((t, er), jnp.float32),
        ],
        compiler_params=pltpu.CompilerParams(
            dimension_semantics=("parallel", "arbitrary"),
        ),
    )(xbf, routing, gate_W, up_W, down_W, bg, bu, ad, ag, au, bd)

    return (out.reshape(b, s, d),
            routing.reshape(b, s, n_e),
            ec.reshape(b, s, n_e))
